# baseline (speedup 1.0000x reference)
import numpy as np
import jax
import jax.numpy as jnp
from jax.experimental import pallas as pl

_EPS = 1e-5


def _pick_th(ho):
    for th in (16, 8, 10, 5):
        if ho % th == 0 and ho // th > 1:
            return th
    return ho


def _conv_kern_factory(TH, Wo, Cin, Cout, stride, relu, has_res):
    Wp = Wo + 2 if stride == 1 else 2 * Wo + 2

    def kern(*refs):
        if has_res:
            r0, r1, r2, w, sb, res, out = refs
        else:
            r0, r1, r2, w, sb, out = refs
        acc = jnp.zeros((TH * Wo, Cout), jnp.float32)
        for dy, rref in enumerate((r0, r1, r2)):
            xb = rref[...]
            if stride == 1:
                xc = jnp.concatenate(
                    [xb[:, 0:Wo, :], xb[:, 1:Wo + 1, :], xb[:, 2:Wo + 2, :]],
                    axis=-1)
            else:
                xr = xb.reshape(TH, Wp // 2, 2, Cin)
                ev = xr[:, :, 0, :]
                od = xr[:, :, 1, :]
                xc = jnp.concatenate(
                    [ev[:, 0:Wo, :], od[:, 0:Wo, :], ev[:, 1:Wo + 1, :]],
                    axis=-1)
            acc = acc + jnp.dot(xc.reshape(TH * Wo, 3 * Cin), w[dy],
                                preferred_element_type=jnp.float32)
        y = (acc.reshape(TH, Wo, Cout) * sb[0:1, :].reshape(1, 1, Cout)
             + sb[1:2, :].reshape(1, 1, Cout))
        if has_res:
            y = y + res[...]
        if relu:
            y = jnp.maximum(y, 0.0)
        out[...] = y

    return kern


def _conv(h, cp, stride=1, bnp=None, relu=False, residual=None):
    H, W, Cin = h.shape
    w = cp['w']
    Cout = w.shape[0]
    wmat = jnp.transpose(w, (2, 3, 1, 0)).reshape(3, 3 * Cin, Cout)
    if bnp is not None:
        s = bnp['g'] / np.sqrt(1.0 + _EPS).astype(np.float32)
        bias = s * cp['b'] + bnp['b']
    else:
        s = jnp.ones((Cout,), jnp.float32)
        bias = cp['b']
    sb = jnp.stack([s, bias])

    xp = jnp.pad(h, ((1, 1), (1, 1), (0, 0)))
    if stride == 1:
        Ho, Wo = H, W
        rows = [xp[dy:dy + Ho] for dy in range(3)]
    else:
        Ho, Wo = H // 2, W // 2
        rows = [xp[dy:dy + 2 * Ho:2] for dy in range(3)]
    Wp = rows[0].shape[1]
    TH = _pick_th(Ho)
    grid = (Ho // TH,)

    row_spec = pl.BlockSpec((TH, Wp, Cin), lambda i: (i, 0, 0))
    in_specs = [row_spec, row_spec, row_spec,
                pl.BlockSpec((3, 3 * Cin, Cout), lambda i: (0, 0, 0)),
                pl.BlockSpec((2, Cout), lambda i: (0, 0))]
    args = rows + [wmat, sb]
    if residual is not None:
        in_specs.append(pl.BlockSpec((TH, Wo, Cout), lambda i: (i, 0, 0)))
        args.append(residual)

    kern = _conv_kern_factory(TH, Wo, Cin, Cout, stride, relu,
                              residual is not None)
    return pl.pallas_call(
        kern,
        grid=grid,
        in_specs=in_specs,
        out_specs=pl.BlockSpec((TH, Wo, Cout), lambda i: (i, 0, 0)),
        out_shape=jax.ShapeDtypeStruct((Ho, Wo, Cout), jnp.float32),
    )(*args)


def _res(h, rp):
    h1 = _conv(h, rp['conv1'], bnp=rp['bn1'], relu=True)
    return _conv(h1, rp['conv2'], bnp=rp['bn2'], residual=h, relu=True)


def _matmul(a, b):
    M, _ = a.shape
    N = b.shape[1]

    def kern(aref, bref, oref):
        oref[...] = jnp.dot(aref[...], bref[...],
                            preferred_element_type=jnp.float32)

    return pl.pallas_call(
        kern,
        out_shape=jax.ShapeDtypeStruct((M, N), jnp.float32),
    )(a, b)


def _interp_matrix(n, m):
    ys = np.linspace(0.0, n - 1.0, m)
    y0 = np.floor(ys).astype(np.int64)
    y1 = np.minimum(y0 + 1, n - 1)
    wy = (ys - y0).astype(np.float32)
    r = np.zeros((m, n), np.float32)
    idx = np.arange(m)
    np.add.at(r, (idx, y0), 1.0 - wy)
    np.add.at(r, (idx, y1), wy)
    return jnp.asarray(r)


def _upsample(h, scale):
    H, W, C = h.shape
    Ho, Wo = H * scale, W * scale
    ry = _interp_matrix(H, Ho)
    rx = _interp_matrix(W, Wo)
    t = _matmul(ry, h.reshape(H, W * C)).reshape(Ho, W, C)
    t = t.transpose(1, 0, 2).reshape(W, Ho * C)
    u = _matmul(rx, t).reshape(Wo, Ho, C).transpose(1, 0, 2)
    return u


def kernel(x, params):
    p = params
    h = jnp.transpose(x[0], (1, 2, 0))
    h = _conv(h, p['conv0'], bnp=p['bn0'], relu=True)
    h = _conv(h, p['conv1'], stride=2)
    h = _res(h, p['res64'])
    h = _conv(h, p['conv2'], stride=2)
    h = _res(h, p['res128'])
    h = _conv(h, p['conv3'], stride=2)
    h = _res(h, p['res256'])
    h = _conv(h, p['conv4'], stride=2)
    h = _res(h, p['res512a'])
    h = _res(h, p['res512b'])
    h = _upsample(h, 2)
    h = _conv(h, p['conv5'])
    h = _upsample(h, 2)
    h = _conv(h, p['conv6'])
    h = _conv(h, p['conv7'], bnp=p['bn7'], relu=True)
    h = _conv(h, p['conv8'])
    h = _upsample(h, 4)
    return jnp.transpose(h, (2, 0, 1))[None]


# speedup vs baseline: 1.4769x; 1.4769x over previous
import numpy as np
import jax
import jax.numpy as jnp
from jax.experimental import pallas as pl
from jax._src.pallas.core import Element as _Element

_EPS = 1e-5


def _pick_th(ho, wp, cin, stride):
    row_bytes = wp * max(cin, 128) * 4
    for th in (50, 25, 20, 16, 10, 8, 5, 4, 2):
        if ho % th:
            continue
        in_rows = th + 2 if stride == 1 else 2 * th + 2
        if in_rows * row_bytes <= 6 * 1024 * 1024:
            return th
    return 1


def _conv_kern_factory(TH, Wo, Cin, Cout, stride, relu, has_res):
    Wp = Wo + 2 if stride == 1 else 2 * Wo + 2

    def kern(*refs):
        if has_res:
            xref, w, sb, res, out = refs
        else:
            xref, w, sb, out = refs
        xb = xref[...]
        acc = jnp.zeros((TH * Wo, Cout), jnp.float32)
        for dy in range(3):
            if stride == 1:
                row = xb[dy:dy + TH]
                xc = jnp.concatenate(
                    [row[:, 0:Wo, :], row[:, 1:Wo + 1, :], row[:, 2:Wo + 2, :]],
                    axis=-1)
            else:
                row = xb[dy:dy + 2 * TH].reshape(TH, 2, Wp, Cin)[:, 0]
                xr = row.reshape(TH, Wp // 2, 2, Cin)
                ev = xr[:, :, 0, :]
                od = xr[:, :, 1, :]
                xc = jnp.concatenate(
                    [ev[:, 0:Wo, :], od[:, 0:Wo, :], ev[:, 1:Wo + 1, :]],
                    axis=-1)
            acc = acc + jnp.dot(xc.reshape(TH * Wo, 3 * Cin), w[dy],
                                preferred_element_type=jnp.float32)
        y = (acc.reshape(TH, Wo, Cout) * sb[0:1, :].reshape(1, 1, Cout)
             + sb[1:2, :].reshape(1, 1, Cout))
        if has_res:
            y = y + res[...]
        if relu:
            y = jnp.maximum(y, 0.0)
        out[...] = y

    return kern


def _conv(h, cp, stride=1, bnp=None, relu=False, residual=None):
    H, W, Cin = h.shape
    w = cp['w']
    Cout = w.shape[0]
    wmat = jnp.transpose(w, (2, 3, 1, 0)).reshape(3, 3 * Cin, Cout)
    if bnp is not None:
        s = bnp['g'] / np.sqrt(1.0 + _EPS).astype(np.float32)
        bias = s * cp['b'] + bnp['b']
    else:
        s = jnp.ones((Cout,), jnp.float32)
        bias = cp['b']
    sb = jnp.stack([s, bias])

    xp = jnp.pad(h, ((1, 1), (1, 1), (0, 0)))
    if stride == 1:
        Ho, Wo = H, W
    else:
        Ho, Wo = H // 2, W // 2
    Wp = xp.shape[1]
    TH = _pick_th(Ho, Wp, Cin, stride)
    grid = (Ho // TH,)
    in_rows = TH + 2 if stride == 1 else 2 * TH + 2
    rstep = TH if stride == 1 else 2 * TH

    x_spec = pl.BlockSpec(
        (_Element(in_rows), _Element(Wp), _Element(Cin)),
        lambda i: (i * rstep, 0, 0))
    in_specs = [x_spec,
                pl.BlockSpec((3, 3 * Cin, Cout), lambda i: (0, 0, 0)),
                pl.BlockSpec((2, Cout), lambda i: (0, 0))]
    args = [xp, wmat, sb]
    if residual is not None:
        in_specs.append(pl.BlockSpec((TH, Wo, Cout), lambda i: (i, 0, 0)))
        args.append(residual)

    kern = _conv_kern_factory(TH, Wo, Cin, Cout, stride, relu,
                              residual is not None)
    return pl.pallas_call(
        kern,
        grid=grid,
        in_specs=in_specs,
        out_specs=pl.BlockSpec((TH, Wo, Cout), lambda i: (i, 0, 0)),
        out_shape=jax.ShapeDtypeStruct((Ho, Wo, Cout), jnp.float32),
    )(*args)


def _res(h, rp):
    h1 = _conv(h, rp['conv1'], bnp=rp['bn1'], relu=True)
    return _conv(h1, rp['conv2'], bnp=rp['bn2'], residual=h, relu=True)


def _matmul(a, b):
    M, _ = a.shape
    N = b.shape[1]

    def kern(aref, bref, oref):
        oref[...] = jnp.dot(aref[...], bref[...],
                            preferred_element_type=jnp.float32)

    return pl.pallas_call(
        kern,
        out_shape=jax.ShapeDtypeStruct((M, N), jnp.float32),
    )(a, b)


def _interp_matrix(n, m):
    ys = np.linspace(0.0, n - 1.0, m)
    y0 = np.floor(ys).astype(np.int64)
    y1 = np.minimum(y0 + 1, n - 1)
    wy = (ys - y0).astype(np.float32)
    r = np.zeros((m, n), np.float32)
    idx = np.arange(m)
    np.add.at(r, (idx, y0), 1.0 - wy)
    np.add.at(r, (idx, y1), wy)
    return jnp.asarray(r)


def _upsample(h, scale):
    H, W, C = h.shape
    Ho, Wo = H * scale, W * scale
    ry = _interp_matrix(H, Ho)
    rx = _interp_matrix(W, Wo)
    t = _matmul(ry, h.reshape(H, W * C)).reshape(Ho, W, C)
    t = t.transpose(1, 0, 2).reshape(W, Ho * C)
    u = _matmul(rx, t).reshape(Wo, Ho, C).transpose(1, 0, 2)
    return u


def kernel(x, params):
    p = params
    h = jnp.transpose(x[0], (1, 2, 0))
    h = _conv(h, p['conv0'], bnp=p['bn0'], relu=True)
    h = _conv(h, p['conv1'], stride=2)
    h = _res(h, p['res64'])
    h = _conv(h, p['conv2'], stride=2)
    h = _res(h, p['res128'])
    h = _conv(h, p['conv3'], stride=2)
    h = _res(h, p['res256'])
    h = _conv(h, p['conv4'], stride=2)
    h = _res(h, p['res512a'])
    h = _res(h, p['res512b'])
    h = _upsample(h, 2)
    h = _conv(h, p['conv5'])
    h = _upsample(h, 2)
    h = _conv(h, p['conv6'])
    h = _conv(h, p['conv7'], bnp=p['bn7'], relu=True)
    h = _conv(h, p['conv8'])
    h = _upsample(h, 4)
    return jnp.transpose(h, (2, 0, 1))[None]


# speedup vs baseline: 1.8447x; 1.2491x over previous
import numpy as np
import jax
import jax.numpy as jnp
from jax.experimental import pallas as pl
from jax._src.pallas.core import Element as _Element

_EPS = 1e-5


def _pick_th(ho, wp, cin, stride):
    row_bytes = wp * max(cin, 128) * 4
    for th in (50, 25, 20, 16, 10, 8, 5, 4, 2):
        if ho % th:
            continue
        in_rows = th + 2 if stride == 1 else 2 * th + 2
        if in_rows * row_bytes <= 6 * 1024 * 1024:
            return th
    return 1


def _conv_kern_factory(TH, Wo, Cin, Cout, stride, relu, has_res, Hin, rstep):
    Wp = Wo + 2 if stride == 1 else 2 * Wo + 2
    in_rows = TH + 2 if stride == 1 else 2 * TH + 2

    def kern(*refs):
        if has_res:
            xref, w, sb, res, out = refs
        else:
            xref, w, sb, out = refs
        i = pl.program_id(0)
        xb = xref[...]
        g = jax.lax.broadcasted_iota(jnp.int32, (in_rows, 1, 1), 0) + i * rstep
        xb = jnp.where((g >= 1) & (g <= Hin), xb, 0.0)
        acc = jnp.zeros((TH * Wo, Cout), jnp.float32)
        for dy in range(3):
            if stride == 1:
                row = xb[dy:dy + TH]
                xc = jnp.concatenate(
                    [row[:, 0:Wo, :], row[:, 1:Wo + 1, :], row[:, 2:Wo + 2, :]],
                    axis=-1)
            else:
                row = xb[dy:dy + 2 * TH].reshape(TH, 2, Wp, Cin)[:, 0]
                xr = row.reshape(TH, Wp // 2, 2, Cin)
                ev = xr[:, :, 0, :]
                od = xr[:, :, 1, :]
                xc = jnp.concatenate(
                    [ev[:, 0:Wo, :], od[:, 0:Wo, :], ev[:, 1:Wo + 1, :]],
                    axis=-1)
            acc = acc + jnp.dot(xc.reshape(TH * Wo, 3 * Cin), w[dy],
                                preferred_element_type=jnp.float32)
        y = (acc.reshape(TH, Wo, Cout) * sb[0:1, :].reshape(1, 1, Cout)
             + sb[1:2, :].reshape(1, 1, Cout))
        if has_res:
            y = y + res[...][:, 1:Wo + 1, :]
        if relu:
            y = jnp.maximum(y, 0.0)
        out[...] = jnp.pad(y, ((0, 0), (1, 1), (0, 0)))

    return kern


def _conv(hp, cp, stride=1, bnp=None, relu=False, residual=None):
    Hin, Win, Cin = hp.shape[0] - 2, hp.shape[1] - 2, hp.shape[2]
    w = cp['w']
    Cout = w.shape[0]
    wmat = jnp.transpose(w, (2, 3, 1, 0)).reshape(3, 3 * Cin, Cout)
    if bnp is not None:
        s = bnp['g'] / np.sqrt(1.0 + _EPS).astype(np.float32)
        bias = s * cp['b'] + bnp['b']
    else:
        s = jnp.ones((Cout,), jnp.float32)
        bias = cp['b']
    sb = jnp.stack([s, bias])

    if stride == 1:
        Ho, Wo = Hin, Win
    else:
        Ho, Wo = Hin // 2, Win // 2
    Wp = hp.shape[1]
    TH = _pick_th(Ho, Wp, Cin, stride)
    grid = (Ho // TH,)
    in_rows = TH + 2 if stride == 1 else 2 * TH + 2
    rstep = TH if stride == 1 else 2 * TH

    x_spec = pl.BlockSpec(
        (_Element(in_rows), _Element(Wp), _Element(Cin)),
        lambda i: (i * rstep, 0, 0))
    in_specs = [x_spec,
                pl.BlockSpec((3, 3 * Cin, Cout), lambda i: (0, 0, 0)),
                pl.BlockSpec((2, Cout), lambda i: (0, 0))]
    args = [hp, wmat, sb]
    if residual is not None:
        in_specs.append(pl.BlockSpec(
            (_Element(TH), _Element(Wo + 2), _Element(Cout)),
            lambda i: (1 + i * TH, 0, 0)))
        args.append(residual)

    kern = _conv_kern_factory(TH, Wo, Cin, Cout, stride, relu,
                              residual is not None, Hin, rstep)
    return pl.pallas_call(
        kern,
        grid=grid,
        in_specs=in_specs,
        out_specs=pl.BlockSpec(
            (_Element(TH), _Element(Wo + 2), _Element(Cout)),
            lambda i: (1 + i * TH, 0, 0)),
        out_shape=jax.ShapeDtypeStruct((Ho + 2, Wo + 2, Cout), jnp.float32),
    )(*args)


def _res(hp, rp):
    h1 = _conv(hp, rp['conv1'], bnp=rp['bn1'], relu=True)
    return _conv(h1, rp['conv2'], bnp=rp['bn2'], residual=hp, relu=True)


def _matmul(a, b):
    M = a.shape[0]
    N = b.shape[1]

    def kern(aref, bref, oref):
        oref[...] = jnp.dot(aref[...], bref[...],
                            preferred_element_type=jnp.float32)

    return pl.pallas_call(
        kern,
        out_shape=jax.ShapeDtypeStruct((M, N), jnp.float32),
    )(a, b)


def _up_row(hp2d, ryp, hin):
    rows = hp2d.shape[0]
    X = hp2d.shape[1]
    Mo = ryp.shape[0]

    def kern(xref, rref, oref):
        xb = xref[...]
        g = jax.lax.broadcasted_iota(jnp.int32, (rows, 1), 0)
        xb = jnp.where((g >= 1) & (g <= hin), xb, 0.0)
        oref[...] = jnp.dot(rref[...], xb, preferred_element_type=jnp.float32)

    return pl.pallas_call(
        kern,
        out_shape=jax.ShapeDtypeStruct((Mo, X), jnp.float32),
    )(hp2d, ryp)


def _interp_matrix(n, m, pad_in=False):
    ys = np.linspace(0.0, n - 1.0, m)
    y0 = np.floor(ys).astype(np.int64)
    y1 = np.minimum(y0 + 1, n - 1)
    wy = (ys - y0).astype(np.float32)
    r = np.zeros((m, n), np.float32)
    idx = np.arange(m)
    np.add.at(r, (idx, y0), 1.0 - wy)
    np.add.at(r, (idx, y1), wy)
    if pad_in:
        r = np.concatenate(
            [np.zeros((m, 1), np.float32), r, np.zeros((m, 1), np.float32)],
            axis=1)
    return jnp.asarray(r)


def _upsample_mid(hp, scale):
    Hp, Wp, C = hp.shape
    H, W = Hp - 2, Wp - 2
    Ho, Wo = H * scale, W * scale
    ryp = _interp_matrix(H, Ho, pad_in=True)
    rxp = _interp_matrix(W, Wo, pad_in=True)
    t = _up_row(hp.reshape(Hp, Wp * C), ryp, H)
    t = t.reshape(Ho, Wp, C).transpose(1, 0, 2).reshape(Wp, Ho * C)
    u = _matmul(rxp, t)
    u = u.reshape(Wo, Ho, C).transpose(1, 0, 2)
    return jnp.pad(u, ((1, 1), (1, 1), (0, 0)))


def _upsample_final(hp, scale):
    Hp, Wp, C = hp.shape
    H, W = Hp - 2, Wp - 2
    Ho, Wo = H * scale, W * scale
    ryp = _interp_matrix(H, Ho, pad_in=True)
    rxpT = jnp.transpose(_interp_matrix(W, Wo, pad_in=True))
    t = _up_row(hp.reshape(Hp, Wp * C), ryp, H)
    t = t.reshape(Ho, Wp, C).transpose(2, 0, 1).reshape(C * Ho, Wp)
    u = _matmul(t, rxpT)
    return u.reshape(1, C, Ho, Wo)


def kernel(x, params):
    p = params
    h = jnp.transpose(jnp.pad(x[0], ((0, 0), (1, 1), (1, 1))), (1, 2, 0))
    h = _conv(h, p['conv0'], bnp=p['bn0'], relu=True)
    h = _conv(h, p['conv1'], stride=2)
    h = _res(h, p['res64'])
    h = _conv(h, p['conv2'], stride=2)
    h = _res(h, p['res128'])
    h = _conv(h, p['conv3'], stride=2)
    h = _res(h, p['res256'])
    h = _conv(h, p['conv4'], stride=2)
    h = _res(h, p['res512a'])
    h = _res(h, p['res512b'])
    h = _upsample_mid(h, 2)
    h = _conv(h, p['conv5'])
    h = _upsample_mid(h, 2)
    h = _conv(h, p['conv6'])
    h = _conv(h, p['conv7'], bnp=p['bn7'], relu=True)
    h = _conv(h, p['conv8'])
    return _upsample_final(h, 4)


# speedup vs baseline: 1.9720x; 1.0690x over previous
import numpy as np
import jax
import jax.numpy as jnp
from jax.experimental import pallas as pl
from jax._src.pallas.core import Element as _Element

_EPS = 1e-5


def _pick_th(ho, wp, cin, stride):
    row_bytes = wp * max(cin, 128) * 4
    for th in (50, 25, 20, 16, 10, 8, 5, 4, 2):
        if ho % th:
            continue
        in_rows = th + 2 if stride == 1 else 2 * th + 2
        if in_rows * row_bytes <= 6 * 1024 * 1024:
            return th
    return 1


def _conv_kern_factory(TH, Wo, Cin, Cout, stride, relu, has_res, Hin, rstep):
    Wp = Wo + 2 if stride == 1 else 2 * Wo + 2
    in_rows = TH + 2 if stride == 1 else 2 * TH + 2

    def kern(*refs):
        if has_res:
            xref, w, sb, res, out = refs
        else:
            xref, w, sb, out = refs
        i = pl.program_id(0)
        xb = xref[...]
        g = jax.lax.broadcasted_iota(jnp.int32, (in_rows, 1, 1), 0) + i * rstep
        xb = jnp.where((g >= 1) & (g <= Hin), xb, 0.0)
        acc = jnp.zeros((TH * Wo, Cout), jnp.float32)
        for dy in range(3):
            if stride == 1:
                row = xb[dy:dy + TH]
                xc = jnp.concatenate(
                    [row[:, 0:Wo, :], row[:, 1:Wo + 1, :], row[:, 2:Wo + 2, :]],
                    axis=-1)
            else:
                row = xb[dy:dy + 2 * TH].reshape(TH, 2, Wp, Cin)[:, 0]
                xr = row.reshape(TH, Wp // 2, 2, Cin)
                ev = xr[:, :, 0, :]
                od = xr[:, :, 1, :]
                xc = jnp.concatenate(
                    [ev[:, 0:Wo, :], od[:, 0:Wo, :], ev[:, 1:Wo + 1, :]],
                    axis=-1)
            acc = acc + jnp.dot(xc.reshape(TH * Wo, 3 * Cin), w[dy],
                                preferred_element_type=jnp.float32)
        y = (acc.reshape(TH, Wo, Cout) * sb[0:1, :].reshape(1, 1, Cout)
             + sb[1:2, :].reshape(1, 1, Cout))
        if has_res:
            y = y + res[...][:, 1:Wo + 1, :]
        if relu:
            y = jnp.maximum(y, 0.0)
        out[...] = jnp.pad(y, ((0, 0), (1, 1), (0, 0)))

    return kern


def _conv(hp, cp, stride=1, bnp=None, relu=False, residual=None):
    Hin, Win, Cin = hp.shape[0] - 2, hp.shape[1] - 2, hp.shape[2]
    w = cp['w']
    Cout = w.shape[0]
    wmat = jnp.transpose(w, (2, 3, 1, 0)).reshape(3, 3 * Cin, Cout)
    if bnp is not None:
        s = bnp['g'] / np.sqrt(1.0 + _EPS).astype(np.float32)
        bias = s * cp['b'] + bnp['b']
    else:
        s = jnp.ones((Cout,), jnp.float32)
        bias = cp['b']
    sb = jnp.stack([s, bias])

    if stride == 1:
        Ho, Wo = Hin, Win
    else:
        Ho, Wo = Hin // 2, Win // 2
    Wp = hp.shape[1]
    TH = _pick_th(Ho, Wp, Cin, stride)
    grid = (Ho // TH,)
    in_rows = TH + 2 if stride == 1 else 2 * TH + 2
    rstep = TH if stride == 1 else 2 * TH

    x_spec = pl.BlockSpec(
        (_Element(in_rows), _Element(Wp), _Element(Cin)),
        lambda i: (i * rstep, 0, 0))
    in_specs = [x_spec,
                pl.BlockSpec((3, 3 * Cin, Cout), lambda i: (0, 0, 0)),
                pl.BlockSpec((2, Cout), lambda i: (0, 0))]
    args = [hp, wmat, sb]
    if residual is not None:
        in_specs.append(pl.BlockSpec(
            (_Element(TH), _Element(Wo + 2), _Element(Cout)),
            lambda i: (1 + i * TH, 0, 0)))
        args.append(residual)

    kern = _conv_kern_factory(TH, Wo, Cin, Cout, stride, relu,
                              residual is not None, Hin, rstep)
    return pl.pallas_call(
        kern,
        grid=grid,
        in_specs=in_specs,
        out_specs=pl.BlockSpec(
            (_Element(TH), _Element(Wo + 2), _Element(Cout)),
            lambda i: (1 + i * TH, 0, 0)),
        out_shape=jax.ShapeDtypeStruct((Ho + 2, Wo + 2, Cout), jnp.float32),
    )(*args)


def _chw_to_hwc_padded(x, th=16):
    C, H, W = x.shape

    def kern(xref, oref):
        y = jnp.transpose(xref[...], (1, 2, 0))
        oref[...] = jnp.pad(y, ((0, 0), (1, 1), (0, 0)))

    return pl.pallas_call(
        kern,
        grid=(H // th,),
        in_specs=[pl.BlockSpec((C, th, W), lambda i: (0, i, 0))],
        out_specs=pl.BlockSpec(
            (_Element(th), _Element(W + 2), _Element(C)),
            lambda i: (1 + i * th, 0, 0)),
        out_shape=jax.ShapeDtypeStruct((H + 2, W + 2, C), jnp.float32),
    )(x)


def _res(hp, rp):
    h1 = _conv(hp, rp['conv1'], bnp=rp['bn1'], relu=True)
    return _conv(h1, rp['conv2'], bnp=rp['bn2'], residual=hp, relu=True)


def _matmul(a, b):
    M = a.shape[0]
    N = b.shape[1]

    def kern(aref, bref, oref):
        oref[...] = jnp.dot(aref[...], bref[...],
                            preferred_element_type=jnp.float32)

    return pl.pallas_call(
        kern,
        out_shape=jax.ShapeDtypeStruct((M, N), jnp.float32),
    )(a, b)


def _up_row(hp2d, ryp, hin):
    rows = hp2d.shape[0]
    X = hp2d.shape[1]
    Mo = ryp.shape[0]

    def kern(xref, rref, oref):
        xb = xref[...]
        g = jax.lax.broadcasted_iota(jnp.int32, (rows, 1), 0)
        xb = jnp.where((g >= 1) & (g <= hin), xb, 0.0)
        oref[...] = jnp.dot(rref[...], xb, preferred_element_type=jnp.float32)

    return pl.pallas_call(
        kern,
        out_shape=jax.ShapeDtypeStruct((Mo, X), jnp.float32),
    )(hp2d, ryp)


def _interp_matrix(n, m, pad_in=False):
    ys = np.linspace(0.0, n - 1.0, m)
    y0 = np.floor(ys).astype(np.int64)
    y1 = np.minimum(y0 + 1, n - 1)
    wy = (ys - y0).astype(np.float32)
    r = np.zeros((m, n), np.float32)
    idx = np.arange(m)
    np.add.at(r, (idx, y0), 1.0 - wy)
    np.add.at(r, (idx, y1), wy)
    if pad_in:
        r = np.concatenate(
            [np.zeros((m, 1), np.float32), r, np.zeros((m, 1), np.float32)],
            axis=1)
    return jnp.asarray(r)


def _upsample_mid(hp, scale):
    Hp, Wp, C = hp.shape
    H, W = Hp - 2, Wp - 2
    Ho, Wo = H * scale, W * scale
    ryp = _interp_matrix(H, Ho, pad_in=True)
    rxp = _interp_matrix(W, Wo, pad_in=True)
    t = _up_row(hp.reshape(Hp, Wp * C), ryp, H)
    t = t.reshape(Ho, Wp, C).transpose(1, 0, 2).reshape(Wp, Ho * C)
    u = _matmul(rxp, t)
    u = u.reshape(Wo, Ho, C).transpose(1, 0, 2)
    return jnp.pad(u, ((1, 1), (1, 1), (0, 0)))


def _upsample_final(hp, scale):
    Hp, Wp, C = hp.shape
    H, W = Hp - 2, Wp - 2
    Ho, Wo = H * scale, W * scale
    ryp = _interp_matrix(H, Ho, pad_in=True)
    rxpT = jnp.transpose(_interp_matrix(W, Wo, pad_in=True))
    t = _up_row(hp.reshape(Hp, Wp * C), ryp, H)
    t = t.reshape(Ho, Wp, C).transpose(2, 0, 1).reshape(C * Ho, Wp)
    u = _matmul(t, rxpT)
    return u.reshape(1, C, Ho, Wo)


def kernel(x, params):
    p = params
    h = _chw_to_hwc_padded(x[0])
    h = _conv(h, p['conv0'], bnp=p['bn0'], relu=True)
    h = _conv(h, p['conv1'], stride=2)
    h = _res(h, p['res64'])
    h = _conv(h, p['conv2'], stride=2)
    h = _res(h, p['res128'])
    h = _conv(h, p['conv3'], stride=2)
    h = _res(h, p['res256'])
    h = _conv(h, p['conv4'], stride=2)
    h = _res(h, p['res512a'])
    h = _res(h, p['res512b'])
    h = _upsample_mid(h, 2)
    h = _conv(h, p['conv5'])
    h = _upsample_mid(h, 2)
    h = _conv(h, p['conv6'])
    h = _conv(h, p['conv7'], bnp=p['bn7'], relu=True)
    h = _conv(h, p['conv8'])
    return _upsample_final(h, 4)


# speedup vs baseline: 2.0681x; 1.0487x over previous
import numpy as np
import jax
import jax.numpy as jnp
from jax.experimental import pallas as pl
from jax._src.pallas.core import Element as _Element

_EPS = 1e-5


def _pick_th(ho, wp, cin, stride):
    row_bytes = wp * max(cin, 128) * 4
    for th in (50, 25, 20, 16, 10, 8, 5, 4, 2):
        if ho % th:
            continue
        in_rows = th + 2 if stride == 1 else 2 * th + 2
        if in_rows * row_bytes <= 6 * 1024 * 1024:
            return th
    return 1


def _conv_kern_factory(TH, Wo, Cin, Cout, stride, relu, has_res, Hin, rstep):
    Wp = Wo + 2 if stride == 1 else 2 * Wo + 2
    in_rows = TH + 2 if stride == 1 else 2 * TH + 2

    def kern(*refs):
        if has_res:
            xref, w, sb, res, out = refs
        else:
            xref, w, sb, out = refs
        i = pl.program_id(0)
        xb = xref[...]
        g = jax.lax.broadcasted_iota(jnp.int32, (in_rows, 1, 1), 0) + i * rstep
        xb = jnp.where((g >= 1) & (g <= Hin), xb, 0.0)
        acc = jnp.zeros((TH * Wo, Cout), jnp.float32)
        for dy in range(3):
            if stride == 1:
                row = xb[dy:dy + TH]
                xc = jnp.concatenate(
                    [row[:, 0:Wo, :], row[:, 1:Wo + 1, :], row[:, 2:Wo + 2, :]],
                    axis=-1)
            else:
                row = xb[dy:dy + 2 * TH].reshape(TH, 2, Wp, Cin)[:, 0]
                xr = row.reshape(TH, Wp // 2, 2, Cin)
                ev = xr[:, :, 0, :]
                od = xr[:, :, 1, :]
                xc = jnp.concatenate(
                    [ev[:, 0:Wo, :], od[:, 0:Wo, :], ev[:, 1:Wo + 1, :]],
                    axis=-1)
            acc = acc + jnp.dot(xc.reshape(TH * Wo, 3 * Cin), w[dy],
                                preferred_element_type=jnp.float32)
        y = (acc.reshape(TH, Wo, Cout) * sb[0:1, :].reshape(1, 1, Cout)
             + sb[1:2, :].reshape(1, 1, Cout))
        if has_res:
            y = y + res[...][:, 1:Wo + 1, :]
        if relu:
            y = jnp.maximum(y, 0.0)
        out[...] = jnp.pad(y, ((0, 0), (1, 1), (0, 0)))

    return kern


def _conv(hp, cp, stride=1, bnp=None, relu=False, residual=None):
    Hin, Win, Cin = hp.shape[0] - 2, hp.shape[1] - 2, hp.shape[2]
    w = cp['w']
    Cout = w.shape[0]
    wmat = jnp.transpose(w, (2, 3, 1, 0)).reshape(3, 3 * Cin, Cout)
    if bnp is not None:
        s = bnp['g'] / np.sqrt(1.0 + _EPS).astype(np.float32)
        bias = s * cp['b'] + bnp['b']
    else:
        s = jnp.ones((Cout,), jnp.float32)
        bias = cp['b']
    sb = jnp.stack([s, bias])

    if stride == 1:
        Ho, Wo = Hin, Win
    else:
        Ho, Wo = Hin // 2, Win // 2
    Wp = hp.shape[1]
    TH = _pick_th(Ho, Wp, Cin, stride)
    grid = (Ho // TH,)
    in_rows = TH + 2 if stride == 1 else 2 * TH + 2
    rstep = TH if stride == 1 else 2 * TH

    x_spec = pl.BlockSpec(
        (_Element(in_rows), _Element(Wp), _Element(Cin)),
        lambda i: (i * rstep, 0, 0))
    in_specs = [x_spec,
                pl.BlockSpec((3, 3 * Cin, Cout), lambda i: (0, 0, 0)),
                pl.BlockSpec((2, Cout), lambda i: (0, 0))]
    args = [hp, wmat, sb]
    if residual is not None:
        in_specs.append(pl.BlockSpec(
            (_Element(TH), _Element(Wo + 2), _Element(Cout)),
            lambda i: (1 + i * TH, 0, 0)))
        args.append(residual)

    kern = _conv_kern_factory(TH, Wo, Cin, Cout, stride, relu,
                              residual is not None, Hin, rstep)
    return pl.pallas_call(
        kern,
        grid=grid,
        in_specs=in_specs,
        out_specs=pl.BlockSpec(
            (_Element(TH), _Element(Wo + 2), _Element(Cout)),
            lambda i: (1 + i * TH, 0, 0)),
        out_shape=jax.ShapeDtypeStruct((Ho + 2, Wo + 2, Cout), jnp.float32),
    )(*args)


def _chw_to_hwc_padded(x, th=16):
    C, H, W = x.shape

    def kern(xref, oref):
        y = jnp.transpose(xref[...], (1, 2, 0))
        oref[...] = jnp.pad(y, ((0, 0), (1, 1), (0, 0)))

    return pl.pallas_call(
        kern,
        grid=(H // th,),
        in_specs=[pl.BlockSpec((C, th, W), lambda i: (0, i, 0))],
        out_specs=pl.BlockSpec(
            (_Element(th), _Element(W + 2), _Element(C)),
            lambda i: (1 + i * th, 0, 0)),
        out_shape=jax.ShapeDtypeStruct((H + 2, W + 2, C), jnp.float32),
    )(x)


def _res(hp, rp):
    h1 = _conv(hp, rp['conv1'], bnp=rp['bn1'], relu=True)
    return _conv(h1, rp['conv2'], bnp=rp['bn2'], residual=hp, relu=True)


def _matmul(a, b):
    M = a.shape[0]
    N = b.shape[1]

    def kern(aref, bref, oref):
        oref[...] = jnp.dot(aref[...], bref[...],
                            preferred_element_type=jnp.float32)

    return pl.pallas_call(
        kern,
        out_shape=jax.ShapeDtypeStruct((M, N), jnp.float32),
    )(a, b)


def _up_row(hp2d, ryp, hin):
    rows = hp2d.shape[0]
    X = hp2d.shape[1]
    Mo = ryp.shape[0]

    def kern(xref, rref, oref):
        xb = xref[...]
        g = jax.lax.broadcasted_iota(jnp.int32, (rows, 1), 0)
        xb = jnp.where((g >= 1) & (g <= hin), xb, 0.0)
        oref[...] = jnp.dot(rref[...], xb, preferred_element_type=jnp.float32)

    return pl.pallas_call(
        kern,
        out_shape=jax.ShapeDtypeStruct((Mo, X), jnp.float32),
    )(hp2d, ryp)


def _interp_matrix(n, m, pad_in=False):
    ys = np.linspace(0.0, n - 1.0, m)
    y0 = np.floor(ys).astype(np.int64)
    y1 = np.minimum(y0 + 1, n - 1)
    wy = (ys - y0).astype(np.float32)
    r = np.zeros((m, n), np.float32)
    idx = np.arange(m)
    np.add.at(r, (idx, y0), 1.0 - wy)
    np.add.at(r, (idx, y1), wy)
    if pad_in:
        r = np.concatenate(
            [np.zeros((m, 1), np.float32), r, np.zeros((m, 1), np.float32)],
            axis=1)
    return jnp.asarray(r)


def _upsample_mid(hp, scale):
    Hp, Wp, C = hp.shape
    H, W = Hp - 2, Wp - 2
    Ho, Wo = H * scale, W * scale
    ryp = _interp_matrix(H, Ho, pad_in=True)
    rxp = _interp_matrix(W, Wo, pad_in=True)
    t = _up_row(hp.reshape(Hp, Wp * C), ryp, H)
    t = t.reshape(Ho, Wp, C).transpose(1, 0, 2).reshape(Wp, Ho * C)

    def kern(tref, rref, oref):
        u = jnp.dot(rref[...], tref[...],
                    preferred_element_type=jnp.float32)
        y = jnp.transpose(u.reshape(Wo, Ho, C), (1, 0, 2))
        oref[...] = jnp.pad(y, ((0, 0), (1, 1), (0, 0)))

    return pl.pallas_call(
        kern,
        grid=(1,),
        in_specs=[pl.BlockSpec(t.shape, lambda i: (0, 0)),
                  pl.BlockSpec(rxp.shape, lambda i: (0, 0))],
        out_specs=pl.BlockSpec(
            (_Element(Ho), _Element(Wo + 2), _Element(C)),
            lambda i: (1, 0, 0)),
        out_shape=jax.ShapeDtypeStruct((Ho + 2, Wo + 2, C), jnp.float32),
    )(t, rxp)


def _upsample_final(hp, scale):
    Hp, Wp, C = hp.shape
    H, W = Hp - 2, Wp - 2
    Ho, Wo = H * scale, W * scale
    ryp = _interp_matrix(H, Ho, pad_in=True)
    rxpT = jnp.transpose(_interp_matrix(W, Wo, pad_in=True))
    t = _up_row(hp.reshape(Hp, Wp * C), ryp, H)
    t = t.reshape(Ho, Wp, C).transpose(2, 0, 1).reshape(C * Ho, Wp)
    u = _matmul(t, rxpT)
    return u.reshape(1, C, Ho, Wo)


def kernel(x, params):
    p = params
    h = _chw_to_hwc_padded(x[0])
    h = _conv(h, p['conv0'], bnp=p['bn0'], relu=True)
    h = _conv(h, p['conv1'], stride=2)
    h = _res(h, p['res64'])
    h = _conv(h, p['conv2'], stride=2)
    h = _res(h, p['res128'])
    h = _conv(h, p['conv3'], stride=2)
    h = _res(h, p['res256'])
    h = _conv(h, p['conv4'], stride=2)
    h = _res(h, p['res512a'])
    h = _res(h, p['res512b'])
    h = _upsample_mid(h, 2)
    h = _conv(h, p['conv5'])
    h = _upsample_mid(h, 2)
    h = _conv(h, p['conv6'])
    h = _conv(h, p['conv7'], bnp=p['bn7'], relu=True)
    h = _conv(h, p['conv8'])
    return _upsample_final(h, 4)
